# baseline (speedup 1.0000x reference)
import functools

import jax
import jax.numpy as jnp
from jax import lax
from jax.experimental import pallas as pl
from jax.experimental.pallas import tpu as pltpu
from jax.experimental.pallas import tpu_sc as plsc

N_PAD = 10240
CHUNK = 128
ROWS = 2560
E_RAW = 320000
PAD_E = ROWS * CHUNK - E_RAW
RPW = ROWS // 32
NCORES = 2
NSUB = 16
STRIPE = N_PAD // NSUB

_mesh = plsc.VectorSubcoreMesh(
    core_axis_name="c", subcore_axis_name="s", num_cores=NCORES, num_subcores=NSUB
)


@functools.partial(
    pl.kernel,
    out_type=jax.ShapeDtypeStruct((NCORES * N_PAD,), jnp.float32),
    mesh=_mesh,
    scratch_types=[
        pltpu.VMEM((128,), jnp.float32),
        pltpu.VMEM((STRIPE,), jnp.float32),
        pltpu.VMEM((RPW, CHUNK), jnp.int32),
        pltpu.VMEM_SHARED((N_PAD,), jnp.float32),
    ],
)
def _deg_kernel(e3_hbm, out_hbm, ones_v, zbuf_v, idx_v, degsh):
    cid = lax.axis_index("c")
    sid = lax.axis_index("s")

    def fill_ones(i, carry):
        ones_v[pl.ds(i * 16, 16)] = jnp.ones((16,), jnp.float32)
        return carry

    lax.fori_loop(0, 128 // 16, fill_ones, 0)

    def fill_zeros(i, carry):
        zbuf_v[pl.ds(i * 16, 16)] = jnp.zeros((16,), jnp.float32)
        return carry

    lax.fori_loop(0, STRIPE // 16, fill_zeros, 0)

    pltpu.sync_copy(zbuf_v, degsh.at[pl.ds(sid * STRIPE, STRIPE)])
    base = cid * (NSUB * RPW) + sid * RPW
    pltpu.sync_copy(e3_hbm.at[0, pl.ds(base, RPW)], idx_v)
    plsc.subcore_barrier()

    def edge_body(j, carry):
        pltpu.sync_copy(
            ones_v.at[pl.ds(0, CHUNK)], degsh.at[idx_v.at[j]], add=True
        )
        return carry

    lax.fori_loop(0, RPW, edge_body, 0)
    plsc.subcore_barrier()
    pltpu.sync_copy(
        degsh.at[pl.ds(sid * STRIPE, STRIPE)],
        out_hbm.at[pl.ds(cid * N_PAD + sid * STRIPE, STRIPE)],
    )


ZR = 64
IB = 40


@functools.partial(
    pl.kernel,
    out_type=jax.ShapeDtypeStruct((NCORES * N_PAD, 128), jnp.float32),
    mesh=_mesh,
    scratch_types=[
        pltpu.VMEM((IB, CHUNK), jnp.int32),
        pltpu.VMEM((IB, CHUNK), jnp.int32),
        pltpu.VMEM((CHUNK, 128), jnp.float32),
        pltpu.VMEM((CHUNK, 128), jnp.float32),
        pltpu.SemaphoreType.DMA,
        pltpu.SemaphoreType.DMA,
        pltpu.VMEM_SHARED((N_PAD, 128), jnp.float32),
    ],
)
def _agg_kernel(hs_hbm, e3_hbm, out_hbm, idxs_v, idxd_v, rows0,
                rows1, sem0, sem1, msgsh):
    cid = lax.axis_index("c")
    sid = lax.axis_index("s")

    def fill_zeros(i, carry):
        r = i // 8
        k = i % 8
        rows0[r, pl.ds(k * 16, 16)] = jnp.zeros((16,), jnp.float32)
        return carry

    lax.fori_loop(0, ZR * 8, fill_zeros, 0)

    def zero_body(t, carry):
        pltpu.sync_copy(
            rows0.at[pl.ds(0, ZR)], msgsh.at[pl.ds(sid * STRIPE + t * ZR, ZR)]
        )
        return carry

    lax.fori_loop(0, STRIPE // ZR, zero_body, 0)

    base = cid * (NSUB * RPW) + sid * RPW
    plsc.subcore_barrier()

    for bi in range(RPW // IB):
        pltpu.sync_copy(e3_hbm.at[0, pl.ds(base + bi * IB, IB)], idxs_v)
        pltpu.sync_copy(e3_hbm.at[1, pl.ds(base + bi * IB, IB)], idxd_v)
        pltpu.async_copy(hs_hbm.at[idxs_v.at[0]], rows0, sem0)

        def pair_body(p, carry):
            pltpu.async_copy(hs_hbm.at[idxs_v.at[2 * p + 1]], rows1, sem1)
            pltpu.make_async_copy(
                hs_hbm.at[idxs_v.at[2 * p]], rows0, sem0
            ).wait()
            pltpu.sync_copy(rows0, msgsh.at[idxd_v.at[2 * p]], add=True)

            @pl.when(p < IB // 2 - 1)
            def _():
                pltpu.async_copy(hs_hbm.at[idxs_v.at[2 * p + 2]], rows0, sem0)

            pltpu.make_async_copy(
                hs_hbm.at[idxs_v.at[2 * p + 1]], rows1, sem1
            ).wait()
            pltpu.sync_copy(rows1, msgsh.at[idxd_v.at[2 * p + 1]], add=True)
            return carry

        lax.fori_loop(0, IB // 2, pair_body, 0)
    plsc.subcore_barrier()
    pltpu.sync_copy(
        msgsh.at[pl.ds(sid * STRIPE, STRIPE)],
        out_hbm.at[pl.ds(cid * N_PAD + sid * STRIPE, STRIPE)],
    )


N = 10000
BLK = 2000


def _lin_body(x_ref, w_ref, b_ref, r_ref, h_ref, hs_ref):
    h = lax.dot_general(
        x_ref[...], w_ref[...], (((1,), (1,)), ((), ())),
        preferred_element_type=jnp.float32,
    ) + b_ref[...]
    h_ref[...] = h
    hs_ref[...] = h * r_ref[...]


def _lin_call(x_p, W, b2, deg3):
    d_out = W.shape[0]
    return pl.pallas_call(
        _lin_body,
        grid=(N // BLK,),
        in_specs=[
            pl.BlockSpec((BLK, x_p.shape[1]), lambda i: (i, 0)),
            pl.BlockSpec(W.shape, lambda i: (0, 0)),
            pl.BlockSpec((1, d_out), lambda i: (0, 0)),
            pl.BlockSpec((BLK, 1), lambda i: (i, 0)),
        ],
        out_specs=[
            pl.BlockSpec((BLK, d_out), lambda i: (i, 0)),
            pl.BlockSpec((BLK, d_out), lambda i: (i, 0)),
        ],
        out_shape=[
            jax.ShapeDtypeStruct((N, d_out), jnp.float32),
            jax.ShapeDtypeStruct((N, d_out), jnp.float32),
        ],
    )(x_p, W, b2, deg3)


def _fin_body(agg_ref, h_ref, r_ref, mask_ref, o_ref):
    msg = (agg_ref[0] + agg_ref[1]) * r_ref[...]
    o_ref[...] = jnp.where(mask_ref[...] != 0.0, msg, h_ref[...])


def _fin_call(agg3, h, deg3, mask2):
    d_out = h.shape[1]
    return pl.pallas_call(
        _fin_body,
        grid=(N // BLK,),
        in_specs=[
            pl.BlockSpec((2, BLK, d_out), lambda i: (0, i, 0)),
            pl.BlockSpec((BLK, d_out), lambda i: (i, 0)),
            pl.BlockSpec((BLK, 1), lambda i: (i, 0)),
            pl.BlockSpec((1, d_out), lambda i: (0, 0)),
        ],
        out_specs=pl.BlockSpec((BLK, d_out), lambda i: (i, 0)),
        out_shape=jax.ShapeDtypeStruct((N, d_out), jnp.float32),
    )(agg3, h, deg3, mask2)


def kernel(x, edge_index, W, b, mask):
    n, _ = x.shape
    d_out = W.shape[0]
    e = edge_index.shape[1]
    assert e == E_RAW and n == N

    pad = jnp.concatenate(
        [jnp.zeros((1, PAD_E), jnp.int32),
         jnp.full((1, PAD_E), N_PAD - 1, jnp.int32)], axis=0)
    e3 = jnp.concatenate([edge_index, pad], axis=1).reshape(2, ROWS, CHUNK)
    degf = _deg_kernel(e3)
    deg = degf.reshape(NCORES, N_PAD).sum(axis=0)
    deg = deg - jnp.where(jnp.arange(N_PAD) == 0, jnp.float32(PAD_E), 0.0)
    r_col = lax.rsqrt(jnp.maximum(deg, 1.0)).reshape(N_PAD, 1)
    h, hs = _lin_call(x, W, b.reshape(1, d_out), r_col)
    aggf = _agg_kernel(hs, e3)
    agg3 = aggf.reshape(NCORES, N_PAD, d_out)
    return _fin_call(agg3, h, r_col, mask.astype(jnp.float32).reshape(1, d_out))


# speedup vs baseline: 3.0941x; 3.0941x over previous
import functools

import jax
import jax.numpy as jnp
from jax import lax
from jax.experimental import pallas as pl
from jax.experimental.pallas import tpu as pltpu
from jax.experimental.pallas import tpu_sc as plsc

N_PAD = 10240
CHUNK = 125
ROWS = 2560
E_RAW = 320000
RPW = ROWS // 32
NCORES = 2
NSUB = 16
STRIPE = N_PAD // NSUB

_mesh = plsc.VectorSubcoreMesh(
    core_axis_name="c", subcore_axis_name="s", num_cores=NCORES, num_subcores=NSUB
)


@functools.partial(
    pl.kernel,
    out_type=jax.ShapeDtypeStruct((NCORES * N_PAD,), jnp.float32),
    mesh=_mesh,
    scratch_types=[
        pltpu.VMEM((128,), jnp.float32),
        pltpu.VMEM((STRIPE,), jnp.float32),
        pltpu.VMEM((RPW, CHUNK), jnp.int32),
        pltpu.VMEM_SHARED((N_PAD,), jnp.float32),
    ],
)
def _deg_kernel(e3_hbm, out_hbm, ones_v, zbuf_v, idx_v, degsh):
    cid = lax.axis_index("c")
    sid = lax.axis_index("s")

    def fill_ones(i, carry):
        ones_v[pl.ds(i * 16, 16)] = jnp.ones((16,), jnp.float32)
        return carry

    lax.fori_loop(0, 128 // 16, fill_ones, 0)

    def fill_zeros(i, carry):
        zbuf_v[pl.ds(i * 16, 16)] = jnp.zeros((16,), jnp.float32)
        return carry

    lax.fori_loop(0, STRIPE // 16, fill_zeros, 0)

    pltpu.sync_copy(zbuf_v, degsh.at[pl.ds(sid * STRIPE, STRIPE)])
    base = cid * (NSUB * RPW) + sid * RPW
    pltpu.sync_copy(e3_hbm.at[0, pl.ds(base, RPW)], idx_v)
    plsc.subcore_barrier()

    def edge_body(j, carry):
        pltpu.sync_copy(
            ones_v.at[pl.ds(0, CHUNK)], degsh.at[idx_v.at[j]], add=True
        )
        return carry

    lax.fori_loop(0, RPW, edge_body, 0)
    plsc.subcore_barrier()
    pltpu.sync_copy(
        degsh.at[pl.ds(sid * STRIPE, STRIPE)],
        out_hbm.at[pl.ds(cid * N_PAD + sid * STRIPE, STRIPE)],
    )


ZR = 64
IB = 40


@functools.partial(
    pl.kernel,
    out_type=jax.ShapeDtypeStruct((NCORES * N_PAD, 128), jnp.float32),
    mesh=_mesh,
    scratch_types=[
        pltpu.VMEM((IB, CHUNK), jnp.int32),
        pltpu.VMEM((IB, CHUNK), jnp.int32),
        pltpu.VMEM((CHUNK, 128), jnp.float32),
        pltpu.VMEM((CHUNK, 128), jnp.float32),
        pltpu.SemaphoreType.DMA,
        pltpu.SemaphoreType.DMA,
        pltpu.VMEM_SHARED((N_PAD, 128), jnp.float32),
    ],
)
def _agg_kernel(hs_hbm, e3_hbm, out_hbm, idxs_v, idxd_v, rows0,
                rows1, sem0, sem1, msgsh):
    cid = lax.axis_index("c")
    sid = lax.axis_index("s")

    def fill_zeros(i, carry):
        r = i // 8
        k = i % 8
        rows0[r, pl.ds(k * 16, 16)] = jnp.zeros((16,), jnp.float32)
        return carry

    lax.fori_loop(0, ZR * 8, fill_zeros, 0)

    def zero_body(t, carry):
        pltpu.sync_copy(
            rows0.at[pl.ds(0, ZR)], msgsh.at[pl.ds(sid * STRIPE + t * ZR, ZR)]
        )
        return carry

    lax.fori_loop(0, STRIPE // ZR, zero_body, 0)

    base = cid * (NSUB * RPW) + sid * RPW
    plsc.subcore_barrier()

    for bi in range(RPW // IB):
        pltpu.sync_copy(e3_hbm.at[0, pl.ds(base + bi * IB, IB)], idxs_v)
        pltpu.sync_copy(e3_hbm.at[1, pl.ds(base + bi * IB, IB)], idxd_v)
        pltpu.async_copy(hs_hbm.at[idxs_v.at[0]], rows0, sem0)

        def pair_body(p, carry):
            pltpu.async_copy(hs_hbm.at[idxs_v.at[2 * p + 1]], rows1, sem1)
            pltpu.make_async_copy(
                hs_hbm.at[idxs_v.at[2 * p]], rows0, sem0
            ).wait()
            pltpu.sync_copy(rows0, msgsh.at[idxd_v.at[2 * p]], add=True)

            @pl.when(p < IB // 2 - 1)
            def _():
                pltpu.async_copy(hs_hbm.at[idxs_v.at[2 * p + 2]], rows0, sem0)

            pltpu.make_async_copy(
                hs_hbm.at[idxs_v.at[2 * p + 1]], rows1, sem1
            ).wait()
            pltpu.sync_copy(rows1, msgsh.at[idxd_v.at[2 * p + 1]], add=True)
            return carry

        lax.fori_loop(0, IB // 2, pair_body, 0)
    plsc.subcore_barrier()
    pltpu.sync_copy(
        msgsh.at[pl.ds(sid * STRIPE, STRIPE)],
        out_hbm.at[pl.ds(cid * N_PAD + sid * STRIPE, STRIPE)],
    )


N = 10000
BLK = 2000


def _lin_body(x_ref, w_ref, b_ref, r_ref, h_ref, hs_ref):
    h = lax.dot_general(
        x_ref[...], w_ref[...], (((1,), (1,)), ((), ())),
        preferred_element_type=jnp.float32,
    ) + b_ref[...]
    h_ref[...] = h
    hs_ref[...] = h * r_ref[...]


def _lin_call(x_p, W, b2, deg3):
    d_out = W.shape[0]
    return pl.pallas_call(
        _lin_body,
        grid=(N // BLK,),
        in_specs=[
            pl.BlockSpec((BLK, x_p.shape[1]), lambda i: (i, 0)),
            pl.BlockSpec(W.shape, lambda i: (0, 0)),
            pl.BlockSpec((1, d_out), lambda i: (0, 0)),
            pl.BlockSpec((BLK, 1), lambda i: (i, 0)),
        ],
        out_specs=[
            pl.BlockSpec((BLK, d_out), lambda i: (i, 0)),
            pl.BlockSpec((BLK, d_out), lambda i: (i, 0)),
        ],
        out_shape=[
            jax.ShapeDtypeStruct((N, d_out), jnp.float32),
            jax.ShapeDtypeStruct((N, d_out), jnp.float32),
        ],
    )(x_p, W, b2, deg3)


def _fin_body(agg_ref, h_ref, r_ref, mask_ref, o_ref):
    msg = (agg_ref[0] + agg_ref[1]) * r_ref[...]
    o_ref[...] = jnp.where(mask_ref[...] != 0.0, msg, h_ref[...])


def _fin_call(agg3, h, deg3, mask2):
    d_out = h.shape[1]
    return pl.pallas_call(
        _fin_body,
        grid=(N // BLK,),
        in_specs=[
            pl.BlockSpec((2, BLK, d_out), lambda i: (0, i, 0)),
            pl.BlockSpec((BLK, d_out), lambda i: (i, 0)),
            pl.BlockSpec((BLK, 1), lambda i: (i, 0)),
            pl.BlockSpec((1, d_out), lambda i: (0, 0)),
        ],
        out_specs=pl.BlockSpec((BLK, d_out), lambda i: (i, 0)),
        out_shape=jax.ShapeDtypeStruct((N, d_out), jnp.float32),
    )(agg3, h, deg3, mask2)


def kernel(x, edge_index, W, b, mask):
    n, _ = x.shape
    d_out = W.shape[0]
    e = edge_index.shape[1]
    assert e == E_RAW and n == N

    e3 = edge_index.reshape(2, ROWS, CHUNK)
    degf = _deg_kernel(e3)
    deg = degf.reshape(NCORES, N_PAD).sum(axis=0)
    r_col = lax.rsqrt(jnp.maximum(deg, 1.0)).reshape(N_PAD, 1)
    h, hs = _lin_call(x, W, b.reshape(1, d_out), r_col)
    aggf = _agg_kernel(hs, e3)
    agg3 = aggf.reshape(NCORES, N_PAD, d_out)
    return _fin_call(agg3, h, r_col, mask.astype(jnp.float32).reshape(1, d_out))


# speedup vs baseline: 3.1914x; 1.0314x over previous
import functools

import jax
import jax.numpy as jnp
from jax import lax
from jax.experimental import pallas as pl
from jax.experimental.pallas import tpu as pltpu
from jax.experimental.pallas import tpu_sc as plsc

N_PAD = 10240
CHUNK = 125
ROWS = 2560
E_RAW = 320000
RPW = ROWS // 32
NCORES = 2
NSUB = 16
STRIPE = N_PAD // NSUB

_mesh = plsc.VectorSubcoreMesh(
    core_axis_name="c", subcore_axis_name="s", num_cores=NCORES, num_subcores=NSUB
)


@functools.partial(
    pl.kernel,
    out_type=jax.ShapeDtypeStruct((NCORES * N_PAD,), jnp.float32),
    mesh=_mesh,
    scratch_types=[
        pltpu.VMEM((128,), jnp.float32),
        pltpu.VMEM((STRIPE,), jnp.float32),
        pltpu.VMEM((RPW, CHUNK), jnp.int32),
        pltpu.SemaphoreType.DMA,
        pltpu.VMEM_SHARED((N_PAD,), jnp.float32),
    ],
)
def _deg_kernel(e3_hbm, out_hbm, ones_v, zbuf_v, idx_v, sem, degsh):
    cid = lax.axis_index("c")
    sid = lax.axis_index("s")

    def fill_ones(i, carry):
        ones_v[pl.ds(i * 16, 16)] = jnp.ones((16,), jnp.float32)
        return carry

    lax.fori_loop(0, 128 // 16, fill_ones, 0)

    def fill_zeros(i, carry):
        zbuf_v[pl.ds(i * 16, 16)] = jnp.zeros((16,), jnp.float32)
        return carry

    lax.fori_loop(0, STRIPE // 16, fill_zeros, 0)

    pltpu.sync_copy(zbuf_v, degsh.at[pl.ds(sid * STRIPE, STRIPE)])
    base = cid * (NSUB * RPW) + sid * RPW
    pltpu.sync_copy(e3_hbm.at[0, pl.ds(base, RPW)], idx_v)
    plsc.subcore_barrier()

    def edge_body(j, carry):
        pltpu.async_copy(
            ones_v.at[pl.ds(0, CHUNK)], degsh.at[idx_v.at[j]], sem, add=True
        )
        return carry

    lax.fori_loop(0, RPW, edge_body, 0)

    def drain_body(j, carry):
        pltpu.make_async_copy(
            ones_v.at[pl.ds(0, CHUNK)], degsh.at[idx_v.at[j]], sem
        ).wait()
        return carry

    lax.fori_loop(0, RPW, drain_body, 0)
    plsc.subcore_barrier()
    pltpu.sync_copy(
        degsh.at[pl.ds(sid * STRIPE, STRIPE)],
        out_hbm.at[pl.ds(cid * N_PAD + sid * STRIPE, STRIPE)],
    )


ZR = 64
IB = 40


@functools.partial(
    pl.kernel,
    out_type=jax.ShapeDtypeStruct((NCORES * N_PAD, 128), jnp.float32),
    mesh=_mesh,
    scratch_types=[
        pltpu.VMEM((IB, CHUNK), jnp.int32),
        pltpu.VMEM((IB, CHUNK), jnp.int32),
        pltpu.VMEM((CHUNK, 128), jnp.float32),
        pltpu.VMEM((CHUNK, 128), jnp.float32),
        pltpu.SemaphoreType.DMA,
        pltpu.SemaphoreType.DMA,
        pltpu.VMEM_SHARED((N_PAD, 128), jnp.float32),
    ],
)
def _agg_kernel(hs_hbm, e3_hbm, out_hbm, idxs_v, idxd_v, rows0,
                rows1, sem0, sem1, msgsh):
    cid = lax.axis_index("c")
    sid = lax.axis_index("s")

    def fill_zeros(i, carry):
        r = i // 8
        k = i % 8
        rows0[r, pl.ds(k * 16, 16)] = jnp.zeros((16,), jnp.float32)
        return carry

    lax.fori_loop(0, ZR * 8, fill_zeros, 0)

    def zero_body(t, carry):
        pltpu.sync_copy(
            rows0.at[pl.ds(0, ZR)], msgsh.at[pl.ds(sid * STRIPE + t * ZR, ZR)]
        )
        return carry

    lax.fori_loop(0, STRIPE // ZR, zero_body, 0)

    base = cid * (NSUB * RPW) + sid * RPW
    plsc.subcore_barrier()

    for bi in range(RPW // IB):
        pltpu.sync_copy(e3_hbm.at[0, pl.ds(base + bi * IB, IB)], idxs_v)
        pltpu.sync_copy(e3_hbm.at[1, pl.ds(base + bi * IB, IB)], idxd_v)
        pltpu.async_copy(hs_hbm.at[idxs_v.at[0]], rows0, sem0)

        def pair_body(p, carry):
            pltpu.async_copy(hs_hbm.at[idxs_v.at[2 * p + 1]], rows1, sem1)
            pltpu.make_async_copy(
                hs_hbm.at[idxs_v.at[2 * p]], rows0, sem0
            ).wait()
            pltpu.sync_copy(rows0, msgsh.at[idxd_v.at[2 * p]], add=True)

            @pl.when(p < IB // 2 - 1)
            def _():
                pltpu.async_copy(hs_hbm.at[idxs_v.at[2 * p + 2]], rows0, sem0)

            pltpu.make_async_copy(
                hs_hbm.at[idxs_v.at[2 * p + 1]], rows1, sem1
            ).wait()
            pltpu.sync_copy(rows1, msgsh.at[idxd_v.at[2 * p + 1]], add=True)
            return carry

        lax.fori_loop(0, IB // 2, pair_body, 0)
    plsc.subcore_barrier()
    pltpu.sync_copy(
        msgsh.at[pl.ds(sid * STRIPE, STRIPE)],
        out_hbm.at[pl.ds(cid * N_PAD + sid * STRIPE, STRIPE)],
    )


N = 10000
BLK = 2000


def _lin_body(x_ref, w_ref, b_ref, r_ref, h_ref, hs_ref):
    h = lax.dot_general(
        x_ref[...], w_ref[...], (((1,), (1,)), ((), ())),
        preferred_element_type=jnp.float32,
    ) + b_ref[...]
    h_ref[...] = h
    hs_ref[...] = h * r_ref[...]


def _lin_call(x_p, W, b2, r_col):
    d_out = W.shape[0]
    return pl.pallas_call(
        _lin_body,
        grid=(N // BLK,),
        in_specs=[
            pl.BlockSpec((BLK, x_p.shape[1]), lambda i: (i, 0)),
            pl.BlockSpec(W.shape, lambda i: (0, 0)),
            pl.BlockSpec((1, d_out), lambda i: (0, 0)),
            pl.BlockSpec((BLK, 1), lambda i: (i, 0)),
        ],
        out_specs=[
            pl.BlockSpec((BLK, d_out), lambda i: (i, 0)),
            pl.BlockSpec((BLK, d_out), lambda i: (i, 0)),
        ],
        out_shape=[
            jax.ShapeDtypeStruct((N, d_out), jnp.float32),
            jax.ShapeDtypeStruct((N, d_out), jnp.float32),
        ],
    )(x_p, W, b2, r_col)


def _fin_body(agg_ref, h_ref, r_ref, mask_ref, o_ref):
    msg = (agg_ref[0] + agg_ref[1]) * r_ref[...]
    o_ref[...] = jnp.where(mask_ref[...] != 0.0, msg, h_ref[...])


def _fin_call(agg3, h, r_col, mask2):
    d_out = h.shape[1]
    return pl.pallas_call(
        _fin_body,
        grid=(N // BLK,),
        in_specs=[
            pl.BlockSpec((2, BLK, d_out), lambda i: (0, i, 0)),
            pl.BlockSpec((BLK, d_out), lambda i: (i, 0)),
            pl.BlockSpec((BLK, 1), lambda i: (i, 0)),
            pl.BlockSpec((1, d_out), lambda i: (0, 0)),
        ],
        out_specs=pl.BlockSpec((BLK, d_out), lambda i: (i, 0)),
        out_shape=jax.ShapeDtypeStruct((N, d_out), jnp.float32),
    )(agg3, h, r_col, mask2)


def kernel(x, edge_index, W, b, mask):
    n, _ = x.shape
    d_out = W.shape[0]
    e = edge_index.shape[1]
    assert e == E_RAW and n == N

    e3 = edge_index.reshape(2, ROWS, CHUNK)
    degf = _deg_kernel(e3)
    deg = degf.reshape(NCORES, N_PAD).sum(axis=0)
    r_col = lax.rsqrt(jnp.maximum(deg, 1.0)).reshape(N_PAD, 1)
    h, hs = _lin_call(x, W, b.reshape(1, d_out), r_col)
    aggf = _agg_kernel(hs, e3)
    agg3 = aggf.reshape(NCORES, N_PAD, d_out)
    return _fin_call(agg3, h, r_col, mask.astype(jnp.float32).reshape(1, d_out))


# speedup vs baseline: 3.2135x; 1.0069x over previous
import functools

import jax
import jax.numpy as jnp
from jax import lax
from jax.experimental import pallas as pl
from jax.experimental.pallas import tpu as pltpu
from jax.experimental.pallas import tpu_sc as plsc

N_PAD = 10240
CHUNK = 125
ROWS = 2560
E_RAW = 320000
RPW = ROWS // 32
NCORES = 2
NSUB = 16
STRIPE = N_PAD // NSUB

_mesh = plsc.VectorSubcoreMesh(
    core_axis_name="c", subcore_axis_name="s", num_cores=NCORES, num_subcores=NSUB
)


@functools.partial(
    pl.kernel,
    out_type=jax.ShapeDtypeStruct((NCORES * N_PAD,), jnp.float32),
    mesh=_mesh,
    scratch_types=[
        pltpu.VMEM((128,), jnp.float32),
        pltpu.VMEM((STRIPE,), jnp.float32),
        pltpu.VMEM((RPW, CHUNK), jnp.int32),
        pltpu.SemaphoreType.DMA,
        pltpu.VMEM_SHARED((N_PAD,), jnp.float32),
    ],
)
def _deg_kernel(e3_hbm, out_hbm, ones_v, zbuf_v, idx_v, sem, degsh):
    cid = lax.axis_index("c")
    sid = lax.axis_index("s")

    def fill_ones(i, carry):
        ones_v[pl.ds(i * 16, 16)] = jnp.ones((16,), jnp.float32)
        return carry

    lax.fori_loop(0, 128 // 16, fill_ones, 0)

    def fill_zeros(i, carry):
        zbuf_v[pl.ds(i * 16, 16)] = jnp.zeros((16,), jnp.float32)
        return carry

    lax.fori_loop(0, STRIPE // 16, fill_zeros, 0)

    pltpu.sync_copy(zbuf_v, degsh.at[pl.ds(sid * STRIPE, STRIPE)])
    base = cid * (NSUB * RPW) + sid * RPW
    pltpu.sync_copy(e3_hbm.at[0, pl.ds(base, RPW)], idx_v)
    plsc.subcore_barrier()

    def edge_body(j, carry):
        pltpu.async_copy(
            ones_v.at[pl.ds(0, CHUNK)], degsh.at[idx_v.at[j]], sem, add=True
        )
        return carry

    lax.fori_loop(0, RPW, edge_body, 0)

    def drain_body(j, carry):
        pltpu.make_async_copy(
            ones_v.at[pl.ds(0, CHUNK)], degsh.at[idx_v.at[j]], sem
        ).wait()
        return carry

    lax.fori_loop(0, RPW, drain_body, 0)
    plsc.subcore_barrier()
    pltpu.sync_copy(
        degsh.at[pl.ds(sid * STRIPE, STRIPE)],
        out_hbm.at[pl.ds(cid * N_PAD + sid * STRIPE, STRIPE)],
    )


ZR = 64
IB = 40


@functools.partial(
    pl.kernel,
    out_type=jax.ShapeDtypeStruct((NCORES * N_PAD, 128), jnp.float32),
    mesh=_mesh,
    scratch_types=[
        pltpu.VMEM((IB, CHUNK), jnp.int32),
        pltpu.VMEM((IB, CHUNK), jnp.int32),
        pltpu.VMEM((CHUNK, 128), jnp.float32),
        pltpu.VMEM((CHUNK, 128), jnp.float32),
        pltpu.SemaphoreType.DMA,
        pltpu.SemaphoreType.DMA,
        pltpu.VMEM_SHARED((N_PAD, 128), jnp.float32),
    ],
)
def _agg_kernel(hs_hbm, e3_hbm, out_hbm, idxs_v, idxd_v, rows0,
                rows1, sem0, sem1, msgsh):
    cid = lax.axis_index("c")
    sid = lax.axis_index("s")
    base = cid * (NSUB * RPW) + sid * RPW

    pltpu.async_copy(e3_hbm.at[0, pl.ds(base, IB)], idxs_v, sem0)
    pltpu.async_copy(e3_hbm.at[1, pl.ds(base, IB)], idxd_v, sem1)

    def fill_zeros(i, carry):
        r = i // 8
        k = i % 8
        rows0[r, pl.ds(k * 16, 16)] = jnp.zeros((16,), jnp.float32)
        return carry

    lax.fori_loop(0, ZR * 8, fill_zeros, 0)

    def zero_body(t, carry):
        pltpu.sync_copy(
            rows0.at[pl.ds(0, ZR)], msgsh.at[pl.ds(sid * STRIPE + t * ZR, ZR)]
        )
        return carry

    lax.fori_loop(0, STRIPE // ZR, zero_body, 0)
    plsc.subcore_barrier()

    for bi in range(RPW // IB):
        if bi == 0:
            pltpu.make_async_copy(
                e3_hbm.at[0, pl.ds(base, IB)], idxs_v, sem0
            ).wait()
            pltpu.make_async_copy(
                e3_hbm.at[1, pl.ds(base, IB)], idxd_v, sem1
            ).wait()
        else:
            pltpu.sync_copy(e3_hbm.at[0, pl.ds(base + bi * IB, IB)], idxs_v)
            pltpu.sync_copy(e3_hbm.at[1, pl.ds(base + bi * IB, IB)], idxd_v)
        pltpu.async_copy(hs_hbm.at[idxs_v.at[0]], rows0, sem0)

        def pair_body(p, carry):
            pltpu.async_copy(hs_hbm.at[idxs_v.at[2 * p + 1]], rows1, sem1)
            pltpu.make_async_copy(
                hs_hbm.at[idxs_v.at[2 * p]], rows0, sem0
            ).wait()
            pltpu.sync_copy(rows0, msgsh.at[idxd_v.at[2 * p]], add=True)

            @pl.when(p < IB // 2 - 1)
            def _():
                pltpu.async_copy(hs_hbm.at[idxs_v.at[2 * p + 2]], rows0, sem0)

            pltpu.make_async_copy(
                hs_hbm.at[idxs_v.at[2 * p + 1]], rows1, sem1
            ).wait()
            pltpu.sync_copy(rows1, msgsh.at[idxd_v.at[2 * p + 1]], add=True)
            return carry

        lax.fori_loop(0, IB // 2, pair_body, 0)
    plsc.subcore_barrier()
    pltpu.sync_copy(
        msgsh.at[pl.ds(sid * STRIPE, STRIPE)],
        out_hbm.at[pl.ds(cid * N_PAD + sid * STRIPE, STRIPE)],
    )


N = 10000
BLK = 2000


def _lin_body(x_ref, w_ref, b_ref, r_ref, h_ref, hs_ref):
    h = lax.dot_general(
        x_ref[...], w_ref[...], (((1,), (1,)), ((), ())),
        preferred_element_type=jnp.float32,
    ) + b_ref[...]
    h_ref[...] = h
    hs_ref[...] = h * r_ref[...]


def _lin_call(x_p, W, b2, r_col):
    d_out = W.shape[0]
    return pl.pallas_call(
        _lin_body,
        grid=(N // BLK,),
        in_specs=[
            pl.BlockSpec((BLK, x_p.shape[1]), lambda i: (i, 0)),
            pl.BlockSpec(W.shape, lambda i: (0, 0)),
            pl.BlockSpec((1, d_out), lambda i: (0, 0)),
            pl.BlockSpec((BLK, 1), lambda i: (i, 0)),
        ],
        out_specs=[
            pl.BlockSpec((BLK, d_out), lambda i: (i, 0)),
            pl.BlockSpec((BLK, d_out), lambda i: (i, 0)),
        ],
        out_shape=[
            jax.ShapeDtypeStruct((N, d_out), jnp.float32),
            jax.ShapeDtypeStruct((N, d_out), jnp.float32),
        ],
    )(x_p, W, b2, r_col)


def _fin_body(agg_ref, h_ref, r_ref, mask_ref, o_ref):
    msg = (agg_ref[0] + agg_ref[1]) * r_ref[...]
    o_ref[...] = jnp.where(mask_ref[...] != 0.0, msg, h_ref[...])


def _fin_call(agg3, h, r_col, mask2):
    d_out = h.shape[1]
    return pl.pallas_call(
        _fin_body,
        grid=(N // BLK,),
        in_specs=[
            pl.BlockSpec((2, BLK, d_out), lambda i: (0, i, 0)),
            pl.BlockSpec((BLK, d_out), lambda i: (i, 0)),
            pl.BlockSpec((BLK, 1), lambda i: (i, 0)),
            pl.BlockSpec((1, d_out), lambda i: (0, 0)),
        ],
        out_specs=pl.BlockSpec((BLK, d_out), lambda i: (i, 0)),
        out_shape=jax.ShapeDtypeStruct((N, d_out), jnp.float32),
    )(agg3, h, r_col, mask2)


def kernel(x, edge_index, W, b, mask):
    n, _ = x.shape
    d_out = W.shape[0]
    e = edge_index.shape[1]
    assert e == E_RAW and n == N

    e3 = edge_index.reshape(2, ROWS, CHUNK)
    degf = _deg_kernel(e3)
    deg = degf.reshape(NCORES, N_PAD).sum(axis=0)
    r_col = lax.rsqrt(jnp.maximum(deg, 1.0)).reshape(N_PAD, 1)
    h, hs = _lin_call(x, W, b.reshape(1, d_out), r_col)
    aggf = _agg_kernel(hs, e3)
    agg3 = aggf.reshape(NCORES, N_PAD, d_out)
    return _fin_call(agg3, h, r_col, mask.astype(jnp.float32).reshape(1, d_out))
